# baseline (speedup 1.0000x reference)
import functools

import jax
import jax.numpy as jnp
from jax import lax
from jax.experimental import pallas as pl
from jax.experimental.pallas import tpu as pltpu
from jax.experimental.pallas import tpu_sc as plsc

N = 10000
E = 320000
D = 128
G = 64

NC = 2
NS = 16
EPT = E // NS
C = 80
NCH = EPT // C
NP = 10240
RPT = NP // NS
RC = 128

_mesh2 = plsc.VectorSubcoreMesh(core_axis_name="c", subcore_axis_name="s")
_mesh1 = plsc.VectorSubcoreMesh(core_axis_name="c", subcore_axis_name="s",
                                num_cores=1)


SCH = 50
NSCH = NCH // SCH


@functools.partial(
    pl.kernel,
    out_type=jax.ShapeDtypeStruct((NC, NP, 16), jnp.float32),
    mesh=_mesh2,
    scratch_types=[
        pltpu.VMEM((SCH, C), jnp.int32),
        pltpu.VMEM((C, 16), jnp.float32),
        pltpu.VMEM_SHARED((NP, 16), jnp.float32),
    ],
)
def _deg_kernel(dsts, z16h, deg_out, dstb, onesb, degsh):
    c = lax.axis_index("c")
    s = lax.axis_index("s")

    ones16 = jnp.full((16,), 1.0, jnp.float32)

    def orow(i, carry):
        onesb[i] = ones16
        return carry

    lax.fori_loop(0, C, orow, 0)

    pltpu.sync_copy(z16h, degsh.at[pl.ds(s * RPT, RPT)])

    plsc.subcore_barrier()

    for sc in range(NSCH):
        pltpu.sync_copy(dsts.at[c, s, sc], dstb)

        def body(i, carry):
            pltpu.sync_copy(onesb, degsh.at[dstb.at[i]], add=True)
            return carry

        lax.fori_loop(0, SCH, body, 0)

    plsc.subcore_barrier()

    pltpu.sync_copy(degsh.at[pl.ds(s * RPT, RPT)],
                    deg_out.at[c, pl.ds(s * RPT, RPT)])


HN = NP // 2
NDUMP = 512
AR = HN + NDUMP
ERC = 64
EPC = HN // NS // ERC


HPT = HN // NS


@functools.partial(
    pl.kernel,
    out_type=(jax.ShapeDtypeStruct((NP, D), jnp.float32),
              jax.ShapeDtypeStruct((NP, D), jnp.float32)),
    mesh=_mesh1,
    scratch_types=[
        pltpu.VMEM((SCH, C), jnp.int32),
        pltpu.VMEM((SCH, C), jnp.int32),
        pltpu.VMEM((2, C, D), jnp.float32),
        pltpu.VMEM_SHARED((AR, D), jnp.float32),
        pltpu.SemaphoreType.DMA,
        pltpu.SemaphoreType.DMA,
        pltpu.SemaphoreType.DMA,
        pltpu.SemaphoreType.DMA,
    ],
)
def _agg_kernel(srcs, dsts, z128, g_t, g_w, acc_t, acc_w,
                srcb, dstb, rows, accsh, sem0, sem1, ssem0, ssem1):
    s = lax.axis_index("s")

    def run_superchunk(g_hbm):
        def gst(i, b, sem):
            return pltpu.make_async_copy(g_hbm.at[srcb.at[i]], rows.at[b],
                                         sem)

        def sst(i, b, sem):
            return pltpu.async_copy(rows.at[b], accsh.at[dstb.at[i]],
                                    sem, add=True)

        gst(0, 0, sem0).start()
        gst(1, 1, sem1).start()

        def body(k, carry):
            i0 = 2 * k
            gst(i0, 0, sem0).wait()
            s0 = sst(i0, 0, ssem0)
            gst(i0 + 1, 1, sem1).wait()
            s1 = sst(i0 + 1, 1, ssem1)
            s0.wait()

            @pl.when(k + 1 < SCH // 2)
            def _():
                gst(i0 + 2, 0, sem0).start()

            s1.wait()

            @pl.when(k + 1 < SCH // 2)
            def _():
                gst(i0 + 3, 1, sem1).start()

            return carry

        lax.fori_loop(0, SCH // 2, body, 0)

    for path, (g_hbm, a_hbm) in enumerate(((g_t, acc_t), (g_w, acc_w))):
        for p in range(2):
            pltpu.sync_copy(z128, accsh.at[pl.ds(s * HPT, HPT)])

            plsc.subcore_barrier()

            for sc in range(NSCH):
                pltpu.sync_copy(srcs.at[path, s, sc], srcb)
                pltpu.sync_copy(dsts.at[path, s, sc], dstb)

                def remap(i, carry):
                    for m in range(C // 16):
                        sl = pl.ds(m * 16, 16)
                        v = dstb[i, sl]
                        in_hi = v >= HN
                        dump = HN + lax.bitwise_and(v, NDUMP - 1)
                        if p == 0:
                            dstb[i, sl] = jnp.where(in_hi, dump, v)
                        else:
                            dstb[i, sl] = jnp.where(in_hi, v - HN, dump)
                    return carry

                lax.fori_loop(0, SCH, remap, 0)

                run_superchunk(g_hbm)

            plsc.subcore_barrier()

            pltpu.sync_copy(accsh.at[pl.ds(s * HPT, HPT)],
                            a_hbm.at[pl.ds(p * HN + s * HPT, HPT)])


_RB = 1000
_NRB = N // _RB

_full = lambda shape: pl.BlockSpec(shape, lambda i: (0,) * len(shape))
_rblk = lambda w: pl.BlockSpec((_RB, w), lambda i: (i, 0))


def _dinv(cnt_blk):
    return lax.rsqrt(cnt_blk + 1.0)


def _tc1_body(x_ref, w1t_ref, w1w_ref, ct_ref, cw_ref, g1t_ref, g1w_ref):
    x = x_ref[...]
    g1t_ref[...] = jnp.dot(x, w1t_ref[...],
                           preferred_element_type=jnp.float32) * _dinv(
                               ct_ref[...])
    g1w_ref[...] = jnp.dot(x, w1w_ref[...],
                           preferred_element_type=jnp.float32) * _dinv(
                               cw_ref[...])


def _tc2_body(at_ref, gt_ref, ct_ref, bt_ref, w2t_ref,
              aw_ref, gw_ref, cw_ref, bw_ref, w2w_ref,
              g2t_ref, g2w_ref):
    for a_ref, g_ref, c_ref, b_ref, w_ref, o_ref in (
            (at_ref, gt_ref, ct_ref, bt_ref, w2t_ref, g2t_ref),
            (aw_ref, gw_ref, cw_ref, bw_ref, w2w_ref, g2w_ref)):
        dv = _dinv(c_ref[...])
        t1 = jax.nn.gelu(dv * (a_ref[...] + g_ref[...]) + b_ref[...])
        o_ref[...] = jnp.dot(t1, w_ref[...],
                             preferred_element_type=jnp.float32) * dv


def _tc3_body(at_ref, gt_ref, ct_ref, bt_ref,
              aw_ref, gw_ref, cw_ref, bw_ref,
              ww_ref, bwf_ref, ids_ref, text_ref, seg_ref):
    dt = _dinv(ct_ref[...])
    text2 = jax.nn.gelu(dt * (at_ref[...] + gt_ref[...]) + bt_ref[...])
    text_ref[...] = text2
    dw = _dinv(cw_ref[...])
    wiki2 = jax.nn.gelu(dw * (aw_ref[...] + gw_ref[...]) + bw_ref[...])
    df = lax.dot_general(text2 - wiki2, ww_ref[...],
                         (((1,), (1,)), ((), ())),
                         preferred_element_type=jnp.float32) + bwf_ref[...]

    @pl.when(pl.program_id(0) == 0)
    def _():
        seg_ref[...] = jnp.full((G, D), -jnp.inf, jnp.float32)

    ids = ids_ref[...]
    neg = jnp.float32(-jnp.inf)
    rows = []
    for gseg in range(G):
        m = ids == gseg
        rows.append(jnp.max(jnp.where(m, df, neg), axis=0, keepdims=True))
    seg_ref[...] = jnp.maximum(seg_ref[...], jnp.concatenate(rows, axis=0))


def kernel(x, edge_index, edge_index_wiki, batch,
           W1t, b1t, W2t, b2t, W1w, b1w, W2w, b2w, Ww, bw):
    f32 = jnp.float32
    srcs = jnp.stack([edge_index[0], edge_index_wiki[0]]).reshape(
        NC, NS, NSCH, SCH, C)
    dsts = jnp.stack([edge_index[1], edge_index_wiki[1]]).reshape(
        NC, NS, NSCH, SCH, C)

    z16h = jnp.zeros((RPT, 16), f32)
    z128 = jnp.zeros((NP // 2 // NS, D), f32)

    deg = _deg_kernel(dsts, z16h)
    cnt_t, cnt_w = deg[0, :, 0:1], deg[1, :, 0:1]

    col = pl.BlockSpec((_RB, 1), lambda i: (i, 0))
    row128 = _full((1, D))
    w128 = _full((D, D))

    g1t, g1w = pl.pallas_call(
        _tc1_body,
        grid=(_NRB,),
        in_specs=[_rblk(D), w128, w128, col, col],
        out_specs=[_rblk(D)] * 2,
        out_shape=[jax.ShapeDtypeStruct((N, D), f32)] * 2,
    )(x, W1t, W1w, cnt_t, cnt_w)

    acc1t, acc1w = _agg_kernel(srcs, dsts, z128, g1t, g1w)

    g2t, g2w = pl.pallas_call(
        _tc2_body,
        grid=(_NRB,),
        in_specs=[_rblk(D), _rblk(D), col, row128, w128,
                  _rblk(D), _rblk(D), col, row128, w128],
        out_specs=[_rblk(D)] * 2,
        out_shape=[jax.ShapeDtypeStruct((N, D), f32)] * 2,
    )(acc1t, g1t, cnt_t, b1t.reshape(1, D), W2t,
      acc1w, g1w, cnt_w, b1w.reshape(1, D), W2w)

    acc2t, acc2w = _agg_kernel(srcs, dsts, z128, g2t, g2w)

    text, seg = pl.pallas_call(
        _tc3_body,
        grid=(_NRB,),
        in_specs=[_rblk(D), _rblk(D), col, row128,
                  _rblk(D), _rblk(D), col, row128,
                  w128, _full((1, D)), col],
        out_specs=[_rblk(D), pl.BlockSpec((G, D), lambda i: (0, 0))],
        out_shape=[jax.ShapeDtypeStruct((N, D), f32),
                   jax.ShapeDtypeStruct((G, D), f32)],
    )(acc2t, g2t, cnt_t, b2t.reshape(1, D),
      acc2w, g2w, cnt_w, b2w.reshape(1, D),
      Ww, bw.reshape(1, D), batch.reshape(N, 1))

    return (text, seg)


# speedup vs baseline: 1.2915x; 1.2915x over previous
import functools

import jax
import jax.numpy as jnp
from jax import lax
from jax.experimental import pallas as pl
from jax.experimental.pallas import tpu as pltpu
from jax.experimental.pallas import tpu_sc as plsc

N = 10000
E = 320000
D = 128
G = 64

NC = 2
NS = 16
EPT = E // NS
C = 80
NCH = EPT // C
NP = 10240
RPT = NP // NS
RC = 128

_mesh2 = plsc.VectorSubcoreMesh(core_axis_name="c", subcore_axis_name="s")
_mesh1 = plsc.VectorSubcoreMesh(core_axis_name="c", subcore_axis_name="s",
                                num_cores=1)


SCH = 50
NSCH = NCH // SCH


@functools.partial(
    pl.kernel,
    out_type=jax.ShapeDtypeStruct((NC, NP, 16), jnp.float32),
    mesh=_mesh2,
    scratch_types=[
        pltpu.VMEM((SCH, C), jnp.int32),
        pltpu.VMEM((C, 16), jnp.float32),
        pltpu.VMEM_SHARED((NP, 16), jnp.float32),
    ],
)
def _deg_kernel(dsts, z16h, deg_out, dstb, onesb, degsh):
    c = lax.axis_index("c")
    s = lax.axis_index("s")

    ones16 = jnp.full((16,), 1.0, jnp.float32)

    def orow(i, carry):
        onesb[i] = ones16
        return carry

    lax.fori_loop(0, C, orow, 0)

    pltpu.sync_copy(z16h, degsh.at[pl.ds(s * RPT, RPT)])

    plsc.subcore_barrier()

    for sc in range(NSCH):
        pltpu.sync_copy(dsts.at[c, s, sc], dstb)

        def body(i, carry):
            pltpu.sync_copy(onesb, degsh.at[dstb.at[i]], add=True)
            return carry

        lax.fori_loop(0, SCH, body, 0)

    plsc.subcore_barrier()

    pltpu.sync_copy(degsh.at[pl.ds(s * RPT, RPT)],
                    deg_out.at[c, pl.ds(s * RPT, RPT)])


HN = NP // 2
NDUMP = 512
AR = HN + NDUMP
ERC = 64
EPC = HN // NS // ERC


HPT = HN // NS


@functools.partial(
    pl.kernel,
    out_type=(jax.ShapeDtypeStruct((NP, D), jnp.float32),
              jax.ShapeDtypeStruct((NP, D), jnp.float32)),
    mesh=_mesh1,
    scratch_types=[
        pltpu.VMEM((SCH, C), jnp.int32),
        pltpu.VMEM((SCH, C), jnp.int32),
        pltpu.VMEM((2, C, D), jnp.float32),
        pltpu.VMEM_SHARED((AR, D), jnp.float32),
        pltpu.SemaphoreType.DMA,
        pltpu.SemaphoreType.DMA,
        pltpu.SemaphoreType.DMA,
        pltpu.SemaphoreType.DMA,
    ],
)
def _agg_kernel(srcs, dsts, z128, g_t, g_w, acc_t, acc_w,
                srcb, dstb, rows, accsh, sem0, sem1, ssem0, ssem1):
    s = lax.axis_index("s")

    def run_superchunk(g_hbm):
        def gst(i, b, sem):
            return pltpu.make_async_copy(g_hbm.at[srcb.at[i]], rows.at[b],
                                         sem)

        gst(0, 0, sem0).start()

        def body(k, carry):
            i0 = 2 * k
            gst(i0 + 1, 1, sem1).start()
            gst(i0, 0, sem0).wait()
            pltpu.sync_copy(rows.at[0], accsh.at[dstb.at[i0]], add=True)

            @pl.when(k + 1 < SCH // 2)
            def _():
                gst(i0 + 2, 0, sem0).start()

            gst(i0 + 1, 1, sem1).wait()
            pltpu.sync_copy(rows.at[1], accsh.at[dstb.at[i0 + 1]], add=True)
            return carry

        lax.fori_loop(0, SCH // 2, body, 0)

    for path, (g_hbm, a_hbm) in enumerate(((g_t, acc_t), (g_w, acc_w))):
        for p in range(2):
            pltpu.sync_copy(z128, accsh.at[pl.ds(s * HPT, HPT)])

            plsc.subcore_barrier()

            for sc in range(NSCH):
                pltpu.sync_copy(srcs.at[path, s, sc], srcb)
                pltpu.sync_copy(dsts.at[path, s, sc], dstb)

                def remap(i, carry):
                    for m in range(C // 16):
                        sl = pl.ds(m * 16, 16)
                        v = dstb[i, sl]
                        in_hi = v >= HN
                        dump = HN + lax.bitwise_and(v, NDUMP - 1)
                        if p == 0:
                            dstb[i, sl] = jnp.where(in_hi, dump, v)
                        else:
                            dstb[i, sl] = jnp.where(in_hi, v - HN, dump)
                    return carry

                lax.fori_loop(0, SCH, remap, 0)

                run_superchunk(g_hbm)

            plsc.subcore_barrier()

            pltpu.sync_copy(accsh.at[pl.ds(s * HPT, HPT)],
                            a_hbm.at[pl.ds(p * HN + s * HPT, HPT)])


_RB = 1000
_NRB = N // _RB

_full = lambda shape: pl.BlockSpec(shape, lambda i: (0,) * len(shape))
_rblk = lambda w: pl.BlockSpec((_RB, w), lambda i: (i, 0))


def _dinv(cnt_blk):
    return lax.rsqrt(cnt_blk + 1.0)


def _tc1_body(x_ref, w1t_ref, w1w_ref, ct_ref, cw_ref, g1t_ref, g1w_ref):
    x = x_ref[...]
    g1t_ref[...] = jnp.dot(x, w1t_ref[...],
                           preferred_element_type=jnp.float32) * _dinv(
                               ct_ref[...])
    g1w_ref[...] = jnp.dot(x, w1w_ref[...],
                           preferred_element_type=jnp.float32) * _dinv(
                               cw_ref[...])


def _tc2_body(at_ref, gt_ref, ct_ref, bt_ref, w2t_ref,
              aw_ref, gw_ref, cw_ref, bw_ref, w2w_ref,
              g2t_ref, g2w_ref):
    for a_ref, g_ref, c_ref, b_ref, w_ref, o_ref in (
            (at_ref, gt_ref, ct_ref, bt_ref, w2t_ref, g2t_ref),
            (aw_ref, gw_ref, cw_ref, bw_ref, w2w_ref, g2w_ref)):
        dv = _dinv(c_ref[...])
        t1 = jax.nn.gelu(dv * (a_ref[...] + g_ref[...]) + b_ref[...])
        o_ref[...] = jnp.dot(t1, w_ref[...],
                             preferred_element_type=jnp.float32) * dv


def _tc3_body(at_ref, gt_ref, ct_ref, bt_ref,
              aw_ref, gw_ref, cw_ref, bw_ref,
              ww_ref, bwf_ref, ids_ref, text_ref, seg_ref):
    dt = _dinv(ct_ref[...])
    text2 = jax.nn.gelu(dt * (at_ref[...] + gt_ref[...]) + bt_ref[...])
    text_ref[...] = text2
    dw = _dinv(cw_ref[...])
    wiki2 = jax.nn.gelu(dw * (aw_ref[...] + gw_ref[...]) + bw_ref[...])
    df = lax.dot_general(text2 - wiki2, ww_ref[...],
                         (((1,), (1,)), ((), ())),
                         preferred_element_type=jnp.float32) + bwf_ref[...]

    @pl.when(pl.program_id(0) == 0)
    def _():
        seg_ref[...] = jnp.full((G, D), -jnp.inf, jnp.float32)

    ids = ids_ref[...]
    neg = jnp.float32(-jnp.inf)
    rows = []
    for gseg in range(G):
        m = ids == gseg
        rows.append(jnp.max(jnp.where(m, df, neg), axis=0, keepdims=True))
    seg_ref[...] = jnp.maximum(seg_ref[...], jnp.concatenate(rows, axis=0))


def kernel(x, edge_index, edge_index_wiki, batch,
           W1t, b1t, W2t, b2t, W1w, b1w, W2w, b2w, Ww, bw):
    f32 = jnp.float32
    srcs = jnp.stack([edge_index[0], edge_index_wiki[0]]).reshape(
        NC, NS, NSCH, SCH, C)
    dsts = jnp.stack([edge_index[1], edge_index_wiki[1]]).reshape(
        NC, NS, NSCH, SCH, C)

    z16h = jnp.zeros((RPT, 16), f32)
    z128 = jnp.zeros((NP // 2 // NS, D), f32)

    deg = _deg_kernel(dsts, z16h)
    cnt_t, cnt_w = deg[0, :, 0:1], deg[1, :, 0:1]

    col = pl.BlockSpec((_RB, 1), lambda i: (i, 0))
    row128 = _full((1, D))
    w128 = _full((D, D))

    g1t, g1w = pl.pallas_call(
        _tc1_body,
        grid=(_NRB,),
        in_specs=[_rblk(D), w128, w128, col, col],
        out_specs=[_rblk(D)] * 2,
        out_shape=[jax.ShapeDtypeStruct((N, D), f32)] * 2,
    )(x, W1t, W1w, cnt_t, cnt_w)

    acc1t, acc1w = _agg_kernel(srcs, dsts, z128, g1t, g1w)

    g2t, g2w = pl.pallas_call(
        _tc2_body,
        grid=(_NRB,),
        in_specs=[_rblk(D), _rblk(D), col, row128, w128,
                  _rblk(D), _rblk(D), col, row128, w128],
        out_specs=[_rblk(D)] * 2,
        out_shape=[jax.ShapeDtypeStruct((N, D), f32)] * 2,
    )(acc1t, g1t, cnt_t, b1t.reshape(1, D), W2t,
      acc1w, g1w, cnt_w, b1w.reshape(1, D), W2w)

    acc2t, acc2w = _agg_kernel(srcs, dsts, z128, g2t, g2w)

    text, seg = pl.pallas_call(
        _tc3_body,
        grid=(_NRB,),
        in_specs=[_rblk(D), _rblk(D), col, row128,
                  _rblk(D), _rblk(D), col, row128,
                  w128, _full((1, D)), col],
        out_specs=[_rblk(D), pl.BlockSpec((G, D), lambda i: (0, 0))],
        out_shape=[jax.ShapeDtypeStruct((N, D), f32),
                   jax.ShapeDtypeStruct((G, D), f32)],
    )(acc2t, g2t, cnt_t, b2t.reshape(1, D),
      acc2w, g2w, cnt_w, b2w.reshape(1, D),
      Ww, bw.reshape(1, D), batch.reshape(N, 1))

    return (text, seg)


# speedup vs baseline: 1.6500x; 1.2775x over previous
import functools

import jax
import jax.numpy as jnp
from jax import lax
from jax.experimental import pallas as pl
from jax.experimental.pallas import tpu as pltpu
from jax.experimental.pallas import tpu_sc as plsc

N = 10000
E = 320000
D = 128
G = 64

NC = 2
NS = 16
EPT = E // NS
C = 80
NCH = EPT // C
NP = 10240
RPT = NP // NS
RC = 128

_mesh2 = plsc.VectorSubcoreMesh(core_axis_name="c", subcore_axis_name="s")
_mesh1 = plsc.VectorSubcoreMesh(core_axis_name="c", subcore_axis_name="s",
                                num_cores=1)


SCH = 50
NSCH = NCH // SCH


@functools.partial(
    pl.kernel,
    out_type=jax.ShapeDtypeStruct((NC, NP, 16), jnp.float32),
    mesh=_mesh2,
    scratch_types=[
        pltpu.VMEM((SCH, C), jnp.int32),
        pltpu.VMEM((C, 16), jnp.float32),
        pltpu.VMEM_SHARED((NP, 16), jnp.float32),
    ],
)
def _deg_kernel(dsts, z16h, deg_out, dstb, onesb, degsh):
    c = lax.axis_index("c")
    s = lax.axis_index("s")

    ones16 = jnp.full((16,), 1.0, jnp.float32)

    def orow(i, carry):
        onesb[i] = ones16
        return carry

    lax.fori_loop(0, C, orow, 0)

    pltpu.sync_copy(z16h, degsh.at[pl.ds(s * RPT, RPT)])

    plsc.subcore_barrier()

    for sc in range(NSCH):
        pltpu.sync_copy(dsts.at[c, s, sc], dstb)

        def body(i, carry):
            pltpu.sync_copy(onesb, degsh.at[dstb.at[i]], add=True)
            return carry

        lax.fori_loop(0, SCH, body, 0)

    plsc.subcore_barrier()

    pltpu.sync_copy(degsh.at[pl.ds(s * RPT, RPT)],
                    deg_out.at[c, pl.ds(s * RPT, RPT)])


HN = NP // 2
NDUMP = 512
AR = HN + NDUMP
ERC = 64
EPC = HN // NS // ERC


HPT = HN // NS


@functools.partial(
    pl.kernel,
    out_type=(jax.ShapeDtypeStruct((NP, D), jnp.float32),
              jax.ShapeDtypeStruct((NP, D), jnp.float32)),
    mesh=_mesh1,
    scratch_types=[
        pltpu.VMEM((SCH, C), jnp.int32),
        pltpu.VMEM((SCH, C), jnp.int32),
        pltpu.VMEM((2, C, D), jnp.float32),
        pltpu.VMEM_SHARED((AR, D), jnp.float32),
        pltpu.SemaphoreType.DMA,
        pltpu.SemaphoreType.DMA,
        pltpu.SemaphoreType.DMA,
        pltpu.SemaphoreType.DMA,
    ],
)
def _agg_kernel(srcs, dsts, z128, g_t, g_w, acc_t, acc_w,
                srcb, dstb, rows, accsh, sem0, sem1, ssem0, ssem1):
    s = lax.axis_index("s")

    def run_superchunk(g_hbm):
        def gst(i, b, sem):
            return pltpu.make_async_copy(g_hbm.at[srcb.at[i]], rows.at[b],
                                         sem)

        gst(0, 0, sem0).start()

        def body(k, carry):
            i0 = 2 * k
            gst(i0 + 1, 1, sem1).start()
            gst(i0, 0, sem0).wait()
            pltpu.sync_copy(rows.at[0], accsh.at[dstb.at[i0]], add=True)

            @pl.when(k + 1 < SCH // 2)
            def _():
                gst(i0 + 2, 0, sem0).start()

            gst(i0 + 1, 1, sem1).wait()
            pltpu.sync_copy(rows.at[1], accsh.at[dstb.at[i0 + 1]], add=True)
            return carry

        lax.fori_loop(0, SCH // 2, body, 0)

    for path, (g_hbm, a_hbm) in enumerate(((g_t, acc_t), (g_w, acc_w))):
        for p in range(2):
            pltpu.sync_copy(z128, accsh.at[pl.ds(s * HPT, HPT)])

            plsc.subcore_barrier()

            for sc in range(NSCH):
                pltpu.sync_copy(srcs.at[path, s, sc], srcb)
                pltpu.sync_copy(dsts.at[path, s, sc], dstb)

                def remap(i, carry):
                    for m in range(C // 16):
                        sl = pl.ds(m * 16, 16)
                        v = dstb[i, sl]
                        in_hi = v >= HN
                        dump = HN + lax.bitwise_and(v, NDUMP - 1)
                        if p == 0:
                            dstb[i, sl] = jnp.where(in_hi, dump, v)
                        else:
                            dstb[i, sl] = jnp.where(in_hi, v - HN, dump)
                    return carry

                lax.fori_loop(0, SCH, remap, 0)

                run_superchunk(g_hbm)

            plsc.subcore_barrier()

            pltpu.sync_copy(accsh.at[pl.ds(s * HPT, HPT)],
                            a_hbm.at[pl.ds(p * HN + s * HPT, HPT)])


@functools.partial(
    pl.kernel,
    out_type=(jax.ShapeDtypeStruct((NP, D), jnp.float32),
              jax.ShapeDtypeStruct((NP, D), jnp.float32)),
    mesh=_mesh1,
    scratch_types=[
        pltpu.VMEM((SCH, C), jnp.int32),
        pltpu.VMEM((SCH, C), jnp.int32),
        pltpu.VMEM((2, C, D), jnp.float32),
        pltpu.VMEM_SHARED((NP, D), jnp.float32),
        pltpu.SemaphoreType.DMA,
        pltpu.SemaphoreType.DMA,
    ],
)
def _agg_full_kernel(srcs, dsts, z128, g_t, g_w, acc_t, acc_w,
                     srcb, dstb, rows, accsh, sem0, sem1):
    s = lax.axis_index("s")

    def run_superchunk(g_hbm):
        def gst(i, b, sem):
            return pltpu.make_async_copy(g_hbm.at[srcb.at[i]], rows.at[b],
                                         sem)

        gst(0, 0, sem0).start()

        def body(k, carry):
            i0 = 2 * k
            gst(i0 + 1, 1, sem1).start()
            gst(i0, 0, sem0).wait()
            pltpu.sync_copy(rows.at[0], accsh.at[dstb.at[i0]], add=True)

            @pl.when(k + 1 < SCH // 2)
            def _():
                gst(i0 + 2, 0, sem0).start()

            gst(i0 + 1, 1, sem1).wait()
            pltpu.sync_copy(rows.at[1], accsh.at[dstb.at[i0 + 1]], add=True)
            return carry

        lax.fori_loop(0, SCH // 2, body, 0)

    for path, (g_hbm, a_hbm) in enumerate(((g_t, acc_t), (g_w, acc_w))):
        pltpu.sync_copy(z128, accsh.at[pl.ds(s * RPT, HPT)])
        pltpu.sync_copy(z128, accsh.at[pl.ds(s * RPT + HPT, HPT)])

        plsc.subcore_barrier()

        for sc in range(NSCH):
            pltpu.sync_copy(srcs.at[path, s, sc], srcb)
            pltpu.sync_copy(dsts.at[path, s, sc], dstb)
            run_superchunk(g_hbm)

        plsc.subcore_barrier()

        pltpu.sync_copy(accsh.at[pl.ds(s * RPT, RPT)],
                        a_hbm.at[pl.ds(s * RPT, RPT)])


_RB = 1000
_NRB = N // _RB

_full = lambda shape: pl.BlockSpec(shape, lambda i: (0,) * len(shape))
_rblk = lambda w: pl.BlockSpec((_RB, w), lambda i: (i, 0))


def _dinv(cnt_blk):
    return lax.rsqrt(cnt_blk + 1.0)


def _tc1_body(x_ref, w1t_ref, w1w_ref, ct_ref, cw_ref, g1t_ref, g1w_ref):
    x = x_ref[...]
    g1t_ref[...] = jnp.dot(x, w1t_ref[...],
                           preferred_element_type=jnp.float32) * _dinv(
                               ct_ref[...])
    g1w_ref[...] = jnp.dot(x, w1w_ref[...],
                           preferred_element_type=jnp.float32) * _dinv(
                               cw_ref[...])


def _tc2_body(at_ref, gt_ref, ct_ref, bt_ref, w2t_ref,
              aw_ref, gw_ref, cw_ref, bw_ref, w2w_ref,
              g2t_ref, g2w_ref):
    for a_ref, g_ref, c_ref, b_ref, w_ref, o_ref in (
            (at_ref, gt_ref, ct_ref, bt_ref, w2t_ref, g2t_ref),
            (aw_ref, gw_ref, cw_ref, bw_ref, w2w_ref, g2w_ref)):
        dv = _dinv(c_ref[...])
        t1 = jax.nn.gelu(dv * (a_ref[...] + g_ref[...]) + b_ref[...])
        o_ref[...] = jnp.dot(t1, w_ref[...],
                             preferred_element_type=jnp.float32) * dv


def _tc3_body(at_ref, gt_ref, ct_ref, bt_ref,
              aw_ref, gw_ref, cw_ref, bw_ref,
              ww_ref, bwf_ref, ids_ref, text_ref, seg_ref):
    dt = _dinv(ct_ref[...])
    text2 = jax.nn.gelu(dt * (at_ref[...] + gt_ref[...]) + bt_ref[...])
    text_ref[...] = text2
    dw = _dinv(cw_ref[...])
    wiki2 = jax.nn.gelu(dw * (aw_ref[...] + gw_ref[...]) + bw_ref[...])
    df = lax.dot_general(text2 - wiki2, ww_ref[...],
                         (((1,), (1,)), ((), ())),
                         preferred_element_type=jnp.float32) + bwf_ref[...]

    @pl.when(pl.program_id(0) == 0)
    def _():
        seg_ref[...] = jnp.full((G, D), -jnp.inf, jnp.float32)

    ids = ids_ref[...]
    neg = jnp.float32(-jnp.inf)
    rows = []
    for gseg in range(G):
        m = ids == gseg
        rows.append(jnp.max(jnp.where(m, df, neg), axis=0, keepdims=True))
    seg_ref[...] = jnp.maximum(seg_ref[...], jnp.concatenate(rows, axis=0))


def kernel(x, edge_index, edge_index_wiki, batch,
           W1t, b1t, W2t, b2t, W1w, b1w, W2w, b2w, Ww, bw):
    f32 = jnp.float32
    srcs = jnp.stack([edge_index[0], edge_index_wiki[0]]).reshape(
        NC, NS, NSCH, SCH, C)
    dsts = jnp.stack([edge_index[1], edge_index_wiki[1]]).reshape(
        NC, NS, NSCH, SCH, C)

    z16h = jnp.zeros((RPT, 16), f32)
    z128 = jnp.zeros((NP // 2 // NS, D), f32)

    deg = _deg_kernel(dsts, z16h)
    cnt_t, cnt_w = deg[0, :, 0:1], deg[1, :, 0:1]

    col = pl.BlockSpec((_RB, 1), lambda i: (i, 0))
    row128 = _full((1, D))
    w128 = _full((D, D))

    g1t, g1w = pl.pallas_call(
        _tc1_body,
        grid=(_NRB,),
        in_specs=[_rblk(D), w128, w128, col, col],
        out_specs=[_rblk(D)] * 2,
        out_shape=[jax.ShapeDtypeStruct((N, D), f32)] * 2,
    )(x, W1t, W1w, cnt_t, cnt_w)

    acc1t, acc1w = _agg_full_kernel(srcs, dsts, z128, g1t, g1w)

    g2t, g2w = pl.pallas_call(
        _tc2_body,
        grid=(_NRB,),
        in_specs=[_rblk(D), _rblk(D), col, row128, w128,
                  _rblk(D), _rblk(D), col, row128, w128],
        out_specs=[_rblk(D)] * 2,
        out_shape=[jax.ShapeDtypeStruct((N, D), f32)] * 2,
    )(acc1t, g1t, cnt_t, b1t.reshape(1, D), W2t,
      acc1w, g1w, cnt_w, b1w.reshape(1, D), W2w)

    acc2t, acc2w = _agg_kernel(srcs, dsts, z128, g2t, g2w)

    text, seg = pl.pallas_call(
        _tc3_body,
        grid=(_NRB,),
        in_specs=[_rblk(D), _rblk(D), col, row128,
                  _rblk(D), _rblk(D), col, row128,
                  w128, _full((1, D)), col],
        out_specs=[_rblk(D), pl.BlockSpec((G, D), lambda i: (0, 0))],
        out_shape=[jax.ShapeDtypeStruct((N, D), f32),
                   jax.ShapeDtypeStruct((G, D), f32)],
    )(acc2t, g2t, cnt_t, b2t.reshape(1, D),
      acc2w, g2w, cnt_w, b2w.reshape(1, D),
      Ww, bw.reshape(1, D), batch.reshape(N, 1))

    return (text, seg)
